# baseline (device time: 45164 ns/iter reference)
import jax
import jax.numpy as jnp
from jax import lax
from jax.experimental import pallas as pl
from jax.experimental.pallas import tpu as pltpu

N_DEV = 4
S = 2


def kernel(A, B):
    m, _ = A.shape
    _, n = B.shape
    nb = n // N_DEV
    ns = nb // S

    def body(a_ref, b_ref, out_ref, part_ref, red_ref, rs_recv, ag_recv,
             rs_send_sems, rs_recv_sems, ag_send_sems, ag_recv_sems):
        my = lax.axis_index("i")

        barrier_sem = pltpu.get_barrier_semaphore()
        for d in range(1, N_DEV):
            pl.semaphore_signal(
                barrier_sem, inc=1,
                device_id=((my + d) % N_DEV,),
                device_id_type=pl.DeviceIdType.MESH,
            )

        a_bf = a_ref[:, :].astype(jnp.bfloat16)

        def rs_send(d, s):
            i = (d - 1) * S + s
            rdma = pltpu.make_async_remote_copy(
                src_ref=part_ref.at[d - 1, :, pl.ds(s * ns, ns)],
                dst_ref=rs_recv.at[d - 1, :, pl.ds(s * ns, ns)],
                send_sem=rs_send_sems.at[i],
                recv_sem=rs_recv_sems.at[i],
                device_id=((my + d) % N_DEV,),
                device_id_type=pl.DeviceIdType.MESH,
            )
            rdma.start()
            return rdma

        rs_rdmas = {}
        for d in range(1, N_DEV):
            g = (my + d) % N_DEV
            b_blk = b_ref[:, pl.ds(g * nb, nb)].astype(jnp.bfloat16)
            part_ref[d - 1, :, :] = jnp.dot(
                a_bf, b_blk, preferred_element_type=jnp.float32
            )
            if d == 1:
                pl.semaphore_wait(barrier_sem, N_DEV - 1)
            for s in range(S):
                rs_rdmas[(d, s)] = rs_send(d, s)

        b_own = b_ref[:, pl.ds(my * nb, nb)].astype(jnp.bfloat16)
        acc = jnp.dot(a_bf, b_own, preferred_element_type=jnp.float32)

        ag_rdmas = {}
        for s in range(S):
            for d in range(1, N_DEV):
                rs_rdmas[(d, s)].wait_recv()
            z = acc[:, s * ns:(s + 1) * ns]
            for k in range(N_DEV - 1):
                z = z + rs_recv[k, :, pl.ds(s * ns, ns)].astype(jnp.float32)
            z = z * (1.0 / (1.0 + jnp.exp(-z)))
            out_ref[:, pl.ds(my * nb + s * ns, ns)] = z
            red_ref[:, pl.ds(s * ns, ns)] = z.astype(jnp.bfloat16)
            for d in range(1, N_DEV):
                i = (d - 1) * S + s
                rdma = pltpu.make_async_remote_copy(
                    src_ref=red_ref.at[:, pl.ds(s * ns, ns)],
                    dst_ref=ag_recv.at[d - 1, :, pl.ds(s * ns, ns)],
                    send_sem=ag_send_sems.at[i],
                    recv_sem=ag_recv_sems.at[i],
                    device_id=((my + d) % N_DEV,),
                    device_id_type=pl.DeviceIdType.MESH,
                )
                rdma.start()
                ag_rdmas[(d, s)] = rdma

        for d in range(1, N_DEV):
            src_dev = (my - d) % N_DEV
            for s in range(S):
                ag_rdmas[(d, s)].wait_recv()
                out_ref[:, pl.ds(src_dev * nb + s * ns, ns)] = (
                    ag_recv[d - 1, :, pl.ds(s * ns, ns)].astype(jnp.float32)
                )

        for r in rs_rdmas.values():
            r.wait_send()
        for r in ag_rdmas.values():
            r.wait_send()

    return pl.pallas_call(
        body,
        out_shape=jax.ShapeDtypeStruct((m, n), jnp.float32),
        in_specs=[
            pl.BlockSpec(memory_space=pltpu.VMEM),
            pl.BlockSpec(memory_space=pltpu.VMEM),
        ],
        out_specs=pl.BlockSpec(memory_space=pltpu.VMEM),
        scratch_shapes=[
            pltpu.VMEM((N_DEV - 1, m, nb), jnp.float32),
            pltpu.VMEM((m, nb), jnp.bfloat16),
            pltpu.VMEM((N_DEV - 1, m, nb), jnp.float32),
            pltpu.VMEM((N_DEV - 1, m, nb), jnp.bfloat16),
            pltpu.SemaphoreType.DMA(((N_DEV - 1) * S,)),
            pltpu.SemaphoreType.DMA(((N_DEV - 1) * S,)),
            pltpu.SemaphoreType.DMA(((N_DEV - 1) * S,)),
            pltpu.SemaphoreType.DMA(((N_DEV - 1) * S,)),
        ],
        compiler_params=pltpu.CompilerParams(collective_id=0),
    )(A, B)


# device time: 33325 ns/iter; 1.3553x vs baseline; 1.3553x over previous
import jax
import jax.numpy as jnp
from jax import lax
from jax.experimental import pallas as pl
from jax.experimental.pallas import tpu as pltpu

N_DEV = 4
S = 2


def kernel(A, B):
    m, _ = A.shape
    _, n = B.shape
    nb = n // N_DEV
    ns = nb // S

    def body(a_ref, b_ref, out_ref, part_ref, red_ref, rs_recv, ag_recv,
             rs_send_sems, rs_recv_sems, ag_send_sems, ag_recv_sems):
        my = lax.axis_index("i")

        barrier_sem = pltpu.get_barrier_semaphore()
        for d in range(1, N_DEV):
            pl.semaphore_signal(
                barrier_sem, inc=1,
                device_id=((my + d) % N_DEV,),
                device_id_type=pl.DeviceIdType.MESH,
            )

        a_bf = a_ref[:, :].astype(jnp.bfloat16)

        def rs_send(d, s):
            i = (d - 1) * S + s
            rdma = pltpu.make_async_remote_copy(
                src_ref=part_ref.at[d - 1, :, pl.ds(s * ns, ns)],
                dst_ref=rs_recv.at[d - 1, :, pl.ds(s * ns, ns)],
                send_sem=rs_send_sems.at[i],
                recv_sem=rs_recv_sems.at[i],
                device_id=((my + d) % N_DEV,),
                device_id_type=pl.DeviceIdType.MESH,
            )
            rdma.start()
            return rdma

        rs_rdmas = {}
        for d in range(1, N_DEV):
            g = (my + d) % N_DEV
            b_blk = b_ref[:, pl.ds(g * nb, nb)].astype(jnp.bfloat16)
            part_ref[d - 1, :, :] = jnp.dot(
                a_bf, b_blk, preferred_element_type=jnp.float32
            ).astype(jnp.bfloat16)
            if d == 1:
                pl.semaphore_wait(barrier_sem, N_DEV - 1)
            for s in range(S):
                rs_rdmas[(d, s)] = rs_send(d, s)

        b_own = b_ref[:, pl.ds(my * nb, nb)].astype(jnp.bfloat16)
        acc = jnp.dot(a_bf, b_own, preferred_element_type=jnp.float32)

        ag_rdmas = {}
        for s in range(S):
            for d in range(1, N_DEV):
                rs_rdmas[(d, s)].wait_recv()
            z = acc[:, s * ns:(s + 1) * ns]
            for k in range(N_DEV - 1):
                z = z + rs_recv[k, :, pl.ds(s * ns, ns)].astype(jnp.float32)
            z = z * (1.0 / (1.0 + jnp.exp(-z)))
            z_bf = z.astype(jnp.bfloat16)
            out_ref[:, pl.ds(my * nb + s * ns, ns)] = z_bf
            red_ref[:, pl.ds(s * ns, ns)] = z_bf
            for d in range(1, N_DEV):
                i = (d - 1) * S + s
                rdma = pltpu.make_async_remote_copy(
                    src_ref=red_ref.at[:, pl.ds(s * ns, ns)],
                    dst_ref=ag_recv.at[d - 1, :, pl.ds(s * ns, ns)],
                    send_sem=ag_send_sems.at[i],
                    recv_sem=ag_recv_sems.at[i],
                    device_id=((my + d) % N_DEV,),
                    device_id_type=pl.DeviceIdType.MESH,
                )
                rdma.start()
                ag_rdmas[(d, s)] = rdma

        for d in range(1, N_DEV):
            src_dev = (my - d) % N_DEV
            for s in range(S):
                ag_rdmas[(d, s)].wait_recv()
                out_ref[:, pl.ds(src_dev * nb + s * ns, ns)] = (
                    ag_recv[d - 1, :, pl.ds(s * ns, ns)]
                )

        for r in rs_rdmas.values():
            r.wait_send()
        for r in ag_rdmas.values():
            r.wait_send()

    return pl.pallas_call(
        body,
        out_shape=jax.ShapeDtypeStruct((m, n), jnp.bfloat16),
        in_specs=[
            pl.BlockSpec(memory_space=pltpu.VMEM),
            pl.BlockSpec(memory_space=pltpu.VMEM),
        ],
        out_specs=pl.BlockSpec(memory_space=pltpu.VMEM),
        scratch_shapes=[
            pltpu.VMEM((N_DEV - 1, m, nb), jnp.bfloat16),
            pltpu.VMEM((m, nb), jnp.bfloat16),
            pltpu.VMEM((N_DEV - 1, m, nb), jnp.bfloat16),
            pltpu.VMEM((N_DEV - 1, m, nb), jnp.bfloat16),
            pltpu.SemaphoreType.DMA(((N_DEV - 1) * S,)),
            pltpu.SemaphoreType.DMA(((N_DEV - 1) * S,)),
            pltpu.SemaphoreType.DMA(((N_DEV - 1) * S,)),
            pltpu.SemaphoreType.DMA(((N_DEV - 1) * S,)),
        ],
        compiler_params=pltpu.CompilerParams(collective_id=0),
    )(A, B)


# device time: 31916 ns/iter; 1.4151x vs baseline; 1.0441x over previous
import jax
import jax.numpy as jnp
from jax import lax
from jax.experimental import pallas as pl
from jax.experimental.pallas import tpu as pltpu

N_DEV = 4
F32 = jnp.float32
BF16 = jnp.bfloat16


def kernel(A, B):
    m, _ = A.shape
    _, n = B.shape
    nb = n // N_DEV
    mh = m // 2

    U = pl.ds(0, mh)
    V = pl.ds(mh, mh)

    def body(a_ref, b_ref, out_ref, part_ref,
             r1y, r1x, c1buf, c2buf, r2x, r2y,
             finu, finv, r3x, r3y, r4y, r4x,
             send_sems, recv_sems):
        my = lax.axis_index("i")
        py = my ^ 1
        px = my ^ 3
        dg = my ^ 2

        def xchg(i, src, dst, peer):
            rdma = pltpu.make_async_remote_copy(
                src_ref=src, dst_ref=dst,
                send_sem=send_sems.at[i], recv_sem=recv_sems.at[i],
                device_id=(peer,), device_id_type=pl.DeviceIdType.MESH,
            )
            rdma.start()
            return rdma

        barrier_sem = pltpu.get_barrier_semaphore()
        for nbr in (py, px):
            pl.semaphore_signal(
                barrier_sem, inc=1,
                device_id=(nbr,), device_id_type=pl.DeviceIdType.MESH,
            )

        a_bf = a_ref[:, :].astype(BF16)

        def pblock(g, slot):
            b_blk = b_ref[:, pl.ds(g * nb, nb)].astype(BF16)
            part_ref[slot, :, :] = jnp.dot(
                a_bf, b_blk, preferred_element_type=F32
            ).astype(BF16)

        pblock(py, 1)
        pl.semaphore_wait(barrier_sem, 2)
        s1y_a = xchg(0, part_ref.at[1, U, :], r1y.at[0], py)
        pblock(px, 3)
        s1x_a = xchg(1, part_ref.at[3, V, :], r1x.at[0], px)
        pblock(dg, 2)
        s1y_b = xchg(2, part_ref.at[2, U, :], r1y.at[1], py)
        s1x_b = xchg(3, part_ref.at[2, V, :], r1x.at[1], px)
        pblock(my, 0)

        s1y_a.wait_recv()
        s1y_b.wait_recv()
        c1_my = part_ref[0, U, :].astype(F32) + r1y[0, :, :].astype(F32)
        c1buf[:, :] = (part_ref[3, U, :].astype(F32)
                       + r1y[1, :, :].astype(F32)).astype(BF16)
        s2x = xchg(4, c1buf.at[:, :], r2x.at[:, :], px)

        s1x_a.wait_recv()
        s1x_b.wait_recv()
        c2_my = part_ref[0, V, :].astype(F32) + r1x[0, :, :].astype(F32)
        c2buf[:, :] = (part_ref[1, V, :].astype(F32)
                       + r1x[1, :, :].astype(F32)).astype(BF16)
        s2y = xchg(5, c2buf.at[:, :], r2y.at[:, :], py)

        def silu(z):
            return z * (1.0 / (1.0 + jnp.exp(-z)))

        s2x.wait_recv()
        fu = silu(c1_my + r2x[:, :].astype(F32)).astype(BF16)
        finu[:, :] = fu
        out_ref[U, pl.ds(my * nb, nb)] = fu
        s3x = xchg(6, finu.at[:, :], r3x.at[:, :], px)

        s2y.wait_recv()
        fv = silu(c2_my + r2y[:, :].astype(F32)).astype(BF16)
        finv[:, :] = fv
        out_ref[V, pl.ds(my * nb, nb)] = fv
        s3y = xchg(7, finv.at[:, :], r3y.at[:, :], py)

        s3x.wait_recv()
        out_ref[U, pl.ds(px * nb, nb)] = r3x[:, :]
        s4y_a = xchg(8, finu.at[:, :], r4y.at[0], py)
        s4y_b = xchg(9, r3x.at[:, :], r4y.at[1], py)

        s3y.wait_recv()
        out_ref[V, pl.ds(py * nb, nb)] = r3y[:, :]
        s4x_a = xchg(10, finv.at[:, :], r4x.at[0], px)
        s4x_b = xchg(11, r3y.at[:, :], r4x.at[1], px)

        s4y_a.wait_recv()
        out_ref[U, pl.ds(py * nb, nb)] = r4y[0, :, :]
        s4y_b.wait_recv()
        out_ref[U, pl.ds(dg * nb, nb)] = r4y[1, :, :]
        s4x_a.wait_recv()
        out_ref[V, pl.ds(px * nb, nb)] = r4x[0, :, :]
        s4x_b.wait_recv()
        out_ref[V, pl.ds(dg * nb, nb)] = r4x[1, :, :]

        for r in (s1y_a, s1y_b, s1x_a, s1x_b, s2x, s2y, s3x, s3y,
                  s4y_a, s4y_b, s4x_a, s4x_b):
            r.wait_send()

    return pl.pallas_call(
        body,
        out_shape=jax.ShapeDtypeStruct((m, n), BF16),
        in_specs=[
            pl.BlockSpec(memory_space=pltpu.VMEM),
            pl.BlockSpec(memory_space=pltpu.VMEM),
        ],
        out_specs=pl.BlockSpec(memory_space=pltpu.VMEM),
        scratch_shapes=[
            pltpu.VMEM((N_DEV, m, nb), BF16),
            pltpu.VMEM((2, mh, nb), BF16),
            pltpu.VMEM((2, mh, nb), BF16),
            pltpu.VMEM((mh, nb), BF16),
            pltpu.VMEM((mh, nb), BF16),
            pltpu.VMEM((mh, nb), BF16),
            pltpu.VMEM((mh, nb), BF16),
            pltpu.VMEM((mh, nb), BF16),
            pltpu.VMEM((mh, nb), BF16),
            pltpu.VMEM((mh, nb), BF16),
            pltpu.VMEM((mh, nb), BF16),
            pltpu.VMEM((2, mh, nb), BF16),
            pltpu.VMEM((2, mh, nb), BF16),
            pltpu.SemaphoreType.DMA((12,)),
            pltpu.SemaphoreType.DMA((12,)),
        ],
        compiler_params=pltpu.CompilerParams(collective_id=0),
    )(A, B)


# device time: 28076 ns/iter; 1.6086x vs baseline; 1.1368x over previous
import jax
import jax.numpy as jnp
from jax import lax
from jax.experimental import pallas as pl
from jax.experimental.pallas import tpu as pltpu

N_DEV = 4
SC = 2
F32 = jnp.float32
BF16 = jnp.bfloat16


def kernel(A, B):
    m, _ = A.shape
    _, n = B.shape
    nb = n // N_DEV
    mh = m // 2
    qr = mh // SC

    def rows_u(sc):
        return pl.ds(sc * qr, qr)

    def rows_v(sc):
        return pl.ds(mh + sc * qr, qr)

    def rows_h(sc):
        return pl.ds(sc * qr, qr)

    def body(a_ref, b_ref, out_ref, part_ref,
             r1y, r1x, c1buf, c2buf, r2x, r2y,
             finu, finv, r3x, r3y, r4y, r4x,
             send_sems, recv_sems):
        my = lax.axis_index("i")
        py = my ^ 1
        px = my ^ 3
        dg = my ^ 2

        sem_i = iter(range(24))
        rdmas = []

        def xchg(src, dst, peer):
            i = next(sem_i)
            rdma = pltpu.make_async_remote_copy(
                src_ref=src, dst_ref=dst,
                send_sem=send_sems.at[i], recv_sem=recv_sems.at[i],
                device_id=(peer,), device_id_type=pl.DeviceIdType.MESH,
            )
            rdma.start()
            rdmas.append(rdma)
            return rdma

        barrier_sem = pltpu.get_barrier_semaphore()
        for nbr in (py, px):
            pl.semaphore_signal(
                barrier_sem, inc=1,
                device_id=(nbr,), device_id_type=pl.DeviceIdType.MESH,
            )

        a_bf = a_ref[:, :].astype(BF16)

        def pblock(g, slot):
            b_blk = b_ref[:, pl.ds(g * nb, nb)].astype(BF16)
            part_ref[slot, :, :] = jnp.dot(
                a_bf, b_blk, preferred_element_type=F32
            ).astype(BF16)

        pblock(py, 1)
        pl.semaphore_wait(barrier_sem, 2)
        s1y_a0 = xchg(part_ref.at[1, rows_u(0), :], r1y.at[0, rows_h(0), :], py)
        pblock(px, 3)
        s1x_a0 = xchg(part_ref.at[3, rows_v(0), :], r1x.at[0, rows_h(0), :], px)
        pblock(dg, 2)
        s1y_b0 = xchg(part_ref.at[2, rows_u(0), :], r1y.at[1, rows_h(0), :], py)
        s1x_b0 = xchg(part_ref.at[2, rows_v(0), :], r1x.at[1, rows_h(0), :], px)
        s1y_a1 = xchg(part_ref.at[1, rows_u(1), :], r1y.at[0, rows_h(1), :], py)
        s1x_a1 = xchg(part_ref.at[3, rows_v(1), :], r1x.at[0, rows_h(1), :], px)
        s1y_b1 = xchg(part_ref.at[2, rows_u(1), :], r1y.at[1, rows_h(1), :], py)
        s1x_b1 = xchg(part_ref.at[2, rows_v(1), :], r1x.at[1, rows_h(1), :], px)
        s1y = [(s1y_a0, s1y_b0), (s1y_a1, s1y_b1)]
        s1x = [(s1x_a0, s1x_b0), (s1x_a1, s1x_b1)]
        pblock(my, 0)

        def silu(z):
            return z * (1.0 / (1.0 + jnp.exp(-z)))

        c1_my, c2_my, s2x, s2y = [None] * SC, [None] * SC, [None] * SC, [None] * SC
        for sc in range(SC):
            a, b = s1y[sc]
            a.wait_recv()
            b.wait_recv()
            c1_my[sc] = (part_ref[0, rows_u(sc), :].astype(F32)
                         + r1y[0, rows_h(sc), :].astype(F32))
            c1buf[rows_h(sc), :] = (part_ref[3, rows_u(sc), :].astype(F32)
                                    + r1y[1, rows_h(sc), :].astype(F32)).astype(BF16)
            s2x[sc] = xchg(c1buf.at[rows_h(sc), :], r2x.at[rows_h(sc), :], px)

            a, b = s1x[sc]
            a.wait_recv()
            b.wait_recv()
            c2_my[sc] = (part_ref[0, rows_v(sc), :].astype(F32)
                         + r1x[0, rows_h(sc), :].astype(F32))
            c2buf[rows_h(sc), :] = (part_ref[1, rows_v(sc), :].astype(F32)
                                    + r1x[1, rows_h(sc), :].astype(F32)).astype(BF16)
            s2y[sc] = xchg(c2buf.at[rows_h(sc), :], r2y.at[rows_h(sc), :], py)

        s3x, s3y = [None] * SC, [None] * SC
        for sc in range(SC):
            s2x[sc].wait_recv()
            fu = silu(c1_my[sc] + r2x[rows_h(sc), :].astype(F32)).astype(BF16)
            finu[rows_h(sc), :] = fu
            out_ref[rows_u(sc), pl.ds(my * nb, nb)] = fu
            s3x[sc] = xchg(finu.at[rows_h(sc), :], r3x.at[rows_h(sc), :], px)

            s2y[sc].wait_recv()
            fv = silu(c2_my[sc] + r2y[rows_h(sc), :].astype(F32)).astype(BF16)
            finv[rows_h(sc), :] = fv
            out_ref[rows_v(sc), pl.ds(my * nb, nb)] = fv
            s3y[sc] = xchg(finv.at[rows_h(sc), :], r3y.at[rows_h(sc), :], py)

        s4y, s4x = [None] * SC, [None] * SC
        for sc in range(SC):
            s3x[sc].wait_recv()
            out_ref[rows_u(sc), pl.ds(px * nb, nb)] = r3x[rows_h(sc), :]
            s4y[sc] = (
                xchg(finu.at[rows_h(sc), :], r4y.at[0, rows_h(sc), :], py),
                xchg(r3x.at[rows_h(sc), :], r4y.at[1, rows_h(sc), :], py),
            )

            s3y[sc].wait_recv()
            out_ref[rows_v(sc), pl.ds(py * nb, nb)] = r3y[rows_h(sc), :]
            s4x[sc] = (
                xchg(finv.at[rows_h(sc), :], r4x.at[0, rows_h(sc), :], px),
                xchg(r3y.at[rows_h(sc), :], r4x.at[1, rows_h(sc), :], px),
            )

        for sc in range(SC):
            a, b = s4y[sc]
            a.wait_recv()
            out_ref[rows_u(sc), pl.ds(py * nb, nb)] = r4y[0, rows_h(sc), :]
            b.wait_recv()
            out_ref[rows_u(sc), pl.ds(dg * nb, nb)] = r4y[1, rows_h(sc), :]
            a, b = s4x[sc]
            a.wait_recv()
            out_ref[rows_v(sc), pl.ds(px * nb, nb)] = r4x[0, rows_h(sc), :]
            b.wait_recv()
            out_ref[rows_v(sc), pl.ds(dg * nb, nb)] = r4x[1, rows_h(sc), :]

        for r in rdmas:
            r.wait_send()

    return pl.pallas_call(
        body,
        out_shape=jax.ShapeDtypeStruct((m, n), BF16),
        in_specs=[
            pl.BlockSpec(memory_space=pltpu.VMEM),
            pl.BlockSpec(memory_space=pltpu.VMEM),
        ],
        out_specs=pl.BlockSpec(memory_space=pltpu.VMEM),
        scratch_shapes=[
            pltpu.VMEM((N_DEV, m, nb), BF16),
            pltpu.VMEM((2, mh, nb), BF16),
            pltpu.VMEM((2, mh, nb), BF16),
            pltpu.VMEM((mh, nb), BF16),
            pltpu.VMEM((mh, nb), BF16),
            pltpu.VMEM((mh, nb), BF16),
            pltpu.VMEM((mh, nb), BF16),
            pltpu.VMEM((mh, nb), BF16),
            pltpu.VMEM((mh, nb), BF16),
            pltpu.VMEM((mh, nb), BF16),
            pltpu.VMEM((mh, nb), BF16),
            pltpu.VMEM((2, mh, nb), BF16),
            pltpu.VMEM((2, mh, nb), BF16),
            pltpu.SemaphoreType.DMA((24,)),
            pltpu.SemaphoreType.DMA((24,)),
        ],
        compiler_params=pltpu.CompilerParams(collective_id=0),
    )(A, B)


# device time: 27868 ns/iter; 1.6206x vs baseline; 1.0075x over previous
import jax
import jax.numpy as jnp
from jax import lax
from jax.experimental import pallas as pl
from jax.experimental.pallas import tpu as pltpu

N_DEV = 4
SC = 4
F32 = jnp.float32
BF16 = jnp.bfloat16


def kernel(A, B):
    m, _ = A.shape
    _, n = B.shape
    nb = n // N_DEV
    mh = m // 2
    qr = mh // SC

    def rows_u(sc):
        return pl.ds(sc * qr, qr)

    def rows_v(sc):
        return pl.ds(mh + sc * qr, qr)

    def rows_h(sc):
        return pl.ds(sc * qr, qr)

    def body(a_ref, b_ref, out_ref, part_ref,
             r1y, r1x, c1buf, c2buf, r2x, r2y,
             finu, finv, r3x, r3y, r4y, r4x,
             send_sems, recv_sems):
        my = lax.axis_index("i")
        py = my ^ 1
        px = my ^ 3
        dg = my ^ 2

        sem_i = iter(range(12 * SC))
        rdmas = []

        def xchg(src, dst, peer):
            i = next(sem_i)
            rdma = pltpu.make_async_remote_copy(
                src_ref=src, dst_ref=dst,
                send_sem=send_sems.at[i], recv_sem=recv_sems.at[i],
                device_id=(peer,), device_id_type=pl.DeviceIdType.MESH,
            )
            rdma.start()
            rdmas.append(rdma)
            return rdma

        barrier_sem = pltpu.get_barrier_semaphore()
        for nbr in (py, px):
            pl.semaphore_signal(
                barrier_sem, inc=1,
                device_id=(nbr,), device_id_type=pl.DeviceIdType.MESH,
            )

        a_bf = a_ref[:, :].astype(BF16)

        def pblock(g, slot):
            b_blk = b_ref[:, pl.ds(g * nb, nb)].astype(BF16)
            part_ref[slot, :, :] = jnp.dot(
                a_bf, b_blk, preferred_element_type=F32
            ).astype(BF16)

        pblock(py, 1)
        pl.semaphore_wait(barrier_sem, 2)
        s1y = [[None, None] for _ in range(SC)]
        s1x = [[None, None] for _ in range(SC)]
        s1y[0][0] = xchg(part_ref.at[1, rows_u(0), :], r1y.at[0, rows_h(0), :], py)
        pblock(px, 3)
        s1x[0][0] = xchg(part_ref.at[3, rows_v(0), :], r1x.at[0, rows_h(0), :], px)
        pblock(dg, 2)
        s1y[0][1] = xchg(part_ref.at[2, rows_u(0), :], r1y.at[1, rows_h(0), :], py)
        s1x[0][1] = xchg(part_ref.at[2, rows_v(0), :], r1x.at[1, rows_h(0), :], px)
        for sc in range(1, SC):
            s1y[sc][0] = xchg(part_ref.at[1, rows_u(sc), :], r1y.at[0, rows_h(sc), :], py)
            s1x[sc][0] = xchg(part_ref.at[3, rows_v(sc), :], r1x.at[0, rows_h(sc), :], px)
            s1y[sc][1] = xchg(part_ref.at[2, rows_u(sc), :], r1y.at[1, rows_h(sc), :], py)
            s1x[sc][1] = xchg(part_ref.at[2, rows_v(sc), :], r1x.at[1, rows_h(sc), :], px)
        pblock(my, 0)

        def silu(z):
            return z * (1.0 / (1.0 + jnp.exp(-z)))

        c1_my, c2_my, s2x, s2y = [None] * SC, [None] * SC, [None] * SC, [None] * SC
        for sc in range(SC):
            a, b = s1y[sc]
            a.wait_recv()
            b.wait_recv()
            c1_my[sc] = (part_ref[0, rows_u(sc), :].astype(F32)
                         + r1y[0, rows_h(sc), :].astype(F32))
            c1buf[rows_h(sc), :] = (part_ref[3, rows_u(sc), :].astype(F32)
                                    + r1y[1, rows_h(sc), :].astype(F32)).astype(BF16)
            s2x[sc] = xchg(c1buf.at[rows_h(sc), :], r2x.at[rows_h(sc), :], px)

            a, b = s1x[sc]
            a.wait_recv()
            b.wait_recv()
            c2_my[sc] = (part_ref[0, rows_v(sc), :].astype(F32)
                         + r1x[0, rows_h(sc), :].astype(F32))
            c2buf[rows_h(sc), :] = (part_ref[1, rows_v(sc), :].astype(F32)
                                    + r1x[1, rows_h(sc), :].astype(F32)).astype(BF16)
            s2y[sc] = xchg(c2buf.at[rows_h(sc), :], r2y.at[rows_h(sc), :], py)

        s3x, s3y = [None] * SC, [None] * SC
        for sc in range(SC):
            s2x[sc].wait_recv()
            fu = silu(c1_my[sc] + r2x[rows_h(sc), :].astype(F32)).astype(BF16)
            finu[rows_h(sc), :] = fu
            out_ref[rows_u(sc), pl.ds(my * nb, nb)] = fu
            s3x[sc] = xchg(finu.at[rows_h(sc), :], r3x.at[rows_h(sc), :], px)

            s2y[sc].wait_recv()
            fv = silu(c2_my[sc] + r2y[rows_h(sc), :].astype(F32)).astype(BF16)
            finv[rows_h(sc), :] = fv
            out_ref[rows_v(sc), pl.ds(my * nb, nb)] = fv
            s3y[sc] = xchg(finv.at[rows_h(sc), :], r3y.at[rows_h(sc), :], py)

        s4y, s4x = [None] * SC, [None] * SC
        for sc in range(SC):
            s3x[sc].wait_recv()
            out_ref[rows_u(sc), pl.ds(px * nb, nb)] = r3x[rows_h(sc), :]
            s4y[sc] = (
                xchg(finu.at[rows_h(sc), :], r4y.at[0, rows_h(sc), :], py),
                xchg(r3x.at[rows_h(sc), :], r4y.at[1, rows_h(sc), :], py),
            )

            s3y[sc].wait_recv()
            out_ref[rows_v(sc), pl.ds(py * nb, nb)] = r3y[rows_h(sc), :]
            s4x[sc] = (
                xchg(finv.at[rows_h(sc), :], r4x.at[0, rows_h(sc), :], px),
                xchg(r3y.at[rows_h(sc), :], r4x.at[1, rows_h(sc), :], px),
            )

        for sc in range(SC):
            a, b = s4y[sc]
            a.wait_recv()
            out_ref[rows_u(sc), pl.ds(py * nb, nb)] = r4y[0, rows_h(sc), :]
            b.wait_recv()
            out_ref[rows_u(sc), pl.ds(dg * nb, nb)] = r4y[1, rows_h(sc), :]
            a, b = s4x[sc]
            a.wait_recv()
            out_ref[rows_v(sc), pl.ds(px * nb, nb)] = r4x[0, rows_h(sc), :]
            b.wait_recv()
            out_ref[rows_v(sc), pl.ds(dg * nb, nb)] = r4x[1, rows_h(sc), :]

        for r in rdmas:
            r.wait_send()

    return pl.pallas_call(
        body,
        out_shape=jax.ShapeDtypeStruct((m, n), BF16),
        in_specs=[
            pl.BlockSpec(memory_space=pltpu.VMEM),
            pl.BlockSpec(memory_space=pltpu.VMEM),
        ],
        out_specs=pl.BlockSpec(memory_space=pltpu.VMEM),
        scratch_shapes=[
            pltpu.VMEM((N_DEV, m, nb), BF16),
            pltpu.VMEM((2, mh, nb), BF16),
            pltpu.VMEM((2, mh, nb), BF16),
            pltpu.VMEM((mh, nb), BF16),
            pltpu.VMEM((mh, nb), BF16),
            pltpu.VMEM((mh, nb), BF16),
            pltpu.VMEM((mh, nb), BF16),
            pltpu.VMEM((mh, nb), BF16),
            pltpu.VMEM((mh, nb), BF16),
            pltpu.VMEM((mh, nb), BF16),
            pltpu.VMEM((mh, nb), BF16),
            pltpu.VMEM((2, mh, nb), BF16),
            pltpu.VMEM((2, mh, nb), BF16),
            pltpu.SemaphoreType.DMA((12 * SC,)),
            pltpu.SemaphoreType.DMA((12 * SC,)),
        ],
        compiler_params=pltpu.CompilerParams(collective_id=0),
    )(A, B)


# device time: 27065 ns/iter; 1.6687x vs baseline; 1.0297x over previous
import jax
import jax.numpy as jnp
from jax import lax
from jax.experimental import pallas as pl
from jax.experimental.pallas import tpu as pltpu

N_DEV = 4
SC = 4
F32 = jnp.float32
BF16 = jnp.bfloat16


def kernel(A, B):
    m, _ = A.shape
    _, n = B.shape
    nb = n // N_DEV
    mh = m // 2
    qr = mh // SC

    def rows_u(sc):
        return pl.ds(sc * qr, qr)

    def rows_v(sc):
        return pl.ds(mh + sc * qr, qr)

    def rows_h(sc):
        return pl.ds(sc * qr, qr)

    def body(a_ref, b_ref, out_ref, part_ref,
             r1y, r1x, c1buf, c2buf, r2x, r2y,
             finu, finv, r3x, r3y, r4y, r4x,
             send_sems, recv_sems):
        my = lax.axis_index("i")
        py = my ^ 1
        px = my ^ 3
        dg = my ^ 2

        sem_i = iter(range(12 * SC))
        rdmas = []

        def xchg(src, dst, peer):
            i = next(sem_i)
            rdma = pltpu.make_async_remote_copy(
                src_ref=src, dst_ref=dst,
                send_sem=send_sems.at[i], recv_sem=recv_sems.at[i],
                device_id=(peer,), device_id_type=pl.DeviceIdType.MESH,
            )
            rdma.start()
            rdmas.append(rdma)
            return rdma

        barrier_sem = pltpu.get_barrier_semaphore()
        for nbr in (py, px):
            pl.semaphore_signal(
                barrier_sem, inc=1,
                device_id=(nbr,), device_id_type=pl.DeviceIdType.MESH,
            )

        a_bf = a_ref[:, :]

        def pblock(g, slot):
            b_blk = b_ref[:, pl.ds(g * nb, nb)]
            part_ref[slot, :, :] = jnp.dot(
                a_bf, b_blk, preferred_element_type=F32
            ).astype(BF16)

        pblock(py, 1)
        pl.semaphore_wait(barrier_sem, 2)
        s1y = [[None, None] for _ in range(SC)]
        s1x = [[None, None] for _ in range(SC)]
        s1y[0][0] = xchg(part_ref.at[1, rows_u(0), :], r1y.at[0, rows_h(0), :], py)
        pblock(px, 3)
        s1x[0][0] = xchg(part_ref.at[3, rows_v(0), :], r1x.at[0, rows_h(0), :], px)
        pblock(dg, 2)
        s1y[0][1] = xchg(part_ref.at[2, rows_u(0), :], r1y.at[1, rows_h(0), :], py)
        s1x[0][1] = xchg(part_ref.at[2, rows_v(0), :], r1x.at[1, rows_h(0), :], px)
        for sc in range(1, SC):
            s1y[sc][0] = xchg(part_ref.at[1, rows_u(sc), :], r1y.at[0, rows_h(sc), :], py)
            s1x[sc][0] = xchg(part_ref.at[3, rows_v(sc), :], r1x.at[0, rows_h(sc), :], px)
            s1y[sc][1] = xchg(part_ref.at[2, rows_u(sc), :], r1y.at[1, rows_h(sc), :], py)
            s1x[sc][1] = xchg(part_ref.at[2, rows_v(sc), :], r1x.at[1, rows_h(sc), :], px)
        pblock(my, 0)

        def silu(z):
            return z * (1.0 / (1.0 + jnp.exp(-z)))

        c1_my, c2_my, s2x, s2y = [None] * SC, [None] * SC, [None] * SC, [None] * SC
        for sc in range(SC):
            a, b = s1y[sc]
            a.wait_recv()
            b.wait_recv()
            c1_my[sc] = (part_ref[0, rows_u(sc), :].astype(F32)
                         + r1y[0, rows_h(sc), :].astype(F32))
            c1buf[rows_h(sc), :] = (part_ref[3, rows_u(sc), :].astype(F32)
                                    + r1y[1, rows_h(sc), :].astype(F32)).astype(BF16)
            s2x[sc] = xchg(c1buf.at[rows_h(sc), :], r2x.at[rows_h(sc), :], px)

            a, b = s1x[sc]
            a.wait_recv()
            b.wait_recv()
            c2_my[sc] = (part_ref[0, rows_v(sc), :].astype(F32)
                         + r1x[0, rows_h(sc), :].astype(F32))
            c2buf[rows_h(sc), :] = (part_ref[1, rows_v(sc), :].astype(F32)
                                    + r1x[1, rows_h(sc), :].astype(F32)).astype(BF16)
            s2y[sc] = xchg(c2buf.at[rows_h(sc), :], r2y.at[rows_h(sc), :], py)

        s3x, s3y = [None] * SC, [None] * SC
        for sc in range(SC):
            s2x[sc].wait_recv()
            fu = silu(c1_my[sc] + r2x[rows_h(sc), :].astype(F32)).astype(BF16)
            finu[rows_h(sc), :] = fu
            out_ref[rows_u(sc), pl.ds(my * nb, nb)] = fu
            s3x[sc] = xchg(finu.at[rows_h(sc), :], r3x.at[rows_h(sc), :], px)

            s2y[sc].wait_recv()
            fv = silu(c2_my[sc] + r2y[rows_h(sc), :].astype(F32)).astype(BF16)
            finv[rows_h(sc), :] = fv
            out_ref[rows_v(sc), pl.ds(my * nb, nb)] = fv
            s3y[sc] = xchg(finv.at[rows_h(sc), :], r3y.at[rows_h(sc), :], py)

        s4y, s4x = [None] * SC, [None] * SC
        for sc in range(SC):
            s3x[sc].wait_recv()
            out_ref[rows_u(sc), pl.ds(px * nb, nb)] = r3x[rows_h(sc), :]
            s4y[sc] = (
                xchg(finu.at[rows_h(sc), :], r4y.at[0, rows_h(sc), :], py),
                xchg(r3x.at[rows_h(sc), :], r4y.at[1, rows_h(sc), :], py),
            )

            s3y[sc].wait_recv()
            out_ref[rows_v(sc), pl.ds(py * nb, nb)] = r3y[rows_h(sc), :]
            s4x[sc] = (
                xchg(finv.at[rows_h(sc), :], r4x.at[0, rows_h(sc), :], px),
                xchg(r3y.at[rows_h(sc), :], r4x.at[1, rows_h(sc), :], px),
            )

        for sc in range(SC):
            a, b = s4y[sc]
            a.wait_recv()
            out_ref[rows_u(sc), pl.ds(py * nb, nb)] = r4y[0, rows_h(sc), :]
            b.wait_recv()
            out_ref[rows_u(sc), pl.ds(dg * nb, nb)] = r4y[1, rows_h(sc), :]
            a, b = s4x[sc]
            a.wait_recv()
            out_ref[rows_v(sc), pl.ds(px * nb, nb)] = r4x[0, rows_h(sc), :]
            b.wait_recv()
            out_ref[rows_v(sc), pl.ds(dg * nb, nb)] = r4x[1, rows_h(sc), :]

        for r in rdmas:
            r.wait_send()

    return pl.pallas_call(
        body,
        out_shape=jax.ShapeDtypeStruct((m, n), BF16),
        in_specs=[
            pl.BlockSpec(memory_space=pltpu.VMEM),
            pl.BlockSpec(memory_space=pltpu.VMEM),
        ],
        out_specs=pl.BlockSpec(memory_space=pltpu.VMEM),
        scratch_shapes=[
            pltpu.VMEM((N_DEV, m, nb), BF16),
            pltpu.VMEM((2, mh, nb), BF16),
            pltpu.VMEM((2, mh, nb), BF16),
            pltpu.VMEM((mh, nb), BF16),
            pltpu.VMEM((mh, nb), BF16),
            pltpu.VMEM((mh, nb), BF16),
            pltpu.VMEM((mh, nb), BF16),
            pltpu.VMEM((mh, nb), BF16),
            pltpu.VMEM((mh, nb), BF16),
            pltpu.VMEM((mh, nb), BF16),
            pltpu.VMEM((mh, nb), BF16),
            pltpu.VMEM((2, mh, nb), BF16),
            pltpu.VMEM((2, mh, nb), BF16),
            pltpu.SemaphoreType.DMA((12 * SC,)),
            pltpu.SemaphoreType.DMA((12 * SC,)),
        ],
        compiler_params=pltpu.CompilerParams(collective_id=0),
    )(A.astype(BF16), B.astype(BF16))


# device time: 27033 ns/iter; 1.6707x vs baseline; 1.0012x over previous
import jax
import jax.numpy as jnp
from jax import lax
from jax.experimental import pallas as pl
from jax.experimental.pallas import tpu as pltpu

N_DEV = 4
SC = 4
F32 = jnp.float32
BF16 = jnp.bfloat16


def kernel(A, B):
    m, _ = A.shape
    _, n = B.shape
    nb = n // N_DEV
    mh = m // 2
    qr = mh // SC

    def rows_u(sc):
        return pl.ds(sc * qr, qr)

    def rows_v(sc):
        return pl.ds(mh + sc * qr, qr)

    def rows_h(sc):
        return pl.ds(sc * qr, qr)

    def body(a_ref, b_ref, out_ref, part_ref,
             r1y, r1x, c1buf, c2buf, r2x, r2y,
             finu, finv, r3x, r3y, r4y, r4x,
             send_sems, recv_sems):
        my = lax.axis_index("i")
        py = my ^ 1
        px = my ^ 3
        dg = my ^ 2

        sem_i = iter(range(12 * SC))
        rdmas = []

        def xchg(src, dst, peer):
            i = next(sem_i)
            rdma = pltpu.make_async_remote_copy(
                src_ref=src, dst_ref=dst,
                send_sem=send_sems.at[i], recv_sem=recv_sems.at[i],
                device_id=(peer,), device_id_type=pl.DeviceIdType.MESH,
            )
            rdma.start()
            rdmas.append(rdma)
            return rdma

        barrier_sem = pltpu.get_barrier_semaphore()
        for nbr in (py, px):
            pl.semaphore_signal(
                barrier_sem, inc=1,
                device_id=(nbr,), device_id_type=pl.DeviceIdType.MESH,
            )

        UH, VH = pl.ds(0, mh), pl.ds(mh, mh)

        def pblock_h(g, slot, half):
            b_blk = b_ref[:, pl.ds(g * nb, nb)]
            part_ref[slot, half, :] = jnp.dot(
                a_ref[half, :], b_blk, preferred_element_type=F32
            ).astype(BF16)

        pblock_h(py, 1, UH)
        pl.semaphore_wait(barrier_sem, 2)
        s1y = [[None, None] for _ in range(SC)]
        s1x = [[None, None] for _ in range(SC)]
        s1y[0][0] = xchg(part_ref.at[1, rows_u(0), :], r1y.at[0, rows_h(0), :], py)
        pblock_h(px, 3, VH)
        s1x[0][0] = xchg(part_ref.at[3, rows_v(0), :], r1x.at[0, rows_h(0), :], px)
        pblock_h(dg, 2, UH)
        s1y[0][1] = xchg(part_ref.at[2, rows_u(0), :], r1y.at[1, rows_h(0), :], py)
        for sc in range(1, SC):
            s1y[sc][0] = xchg(part_ref.at[1, rows_u(sc), :], r1y.at[0, rows_h(sc), :], py)
            s1y[sc][1] = xchg(part_ref.at[2, rows_u(sc), :], r1y.at[1, rows_h(sc), :], py)
        pblock_h(dg, 2, VH)
        s1x[0][1] = xchg(part_ref.at[2, rows_v(0), :], r1x.at[1, rows_h(0), :], px)
        for sc in range(1, SC):
            s1x[sc][0] = xchg(part_ref.at[3, rows_v(sc), :], r1x.at[0, rows_h(sc), :], px)
            s1x[sc][1] = xchg(part_ref.at[2, rows_v(sc), :], r1x.at[1, rows_h(sc), :], px)
        pblock_h(px, 3, UH)
        pblock_h(py, 1, VH)
        pblock_h(my, 0, UH)
        pblock_h(my, 0, VH)

        def silu(z):
            return z * (1.0 / (1.0 + jnp.exp(-z)))

        c1_my, c2_my, s2x, s2y = [None] * SC, [None] * SC, [None] * SC, [None] * SC
        for sc in range(SC):
            a, b = s1y[sc]
            a.wait_recv()
            b.wait_recv()
            c1_my[sc] = (part_ref[0, rows_u(sc), :].astype(F32)
                         + r1y[0, rows_h(sc), :].astype(F32))
            c1buf[rows_h(sc), :] = (part_ref[3, rows_u(sc), :].astype(F32)
                                    + r1y[1, rows_h(sc), :].astype(F32)).astype(BF16)
            s2x[sc] = xchg(c1buf.at[rows_h(sc), :], r2x.at[rows_h(sc), :], px)

            a, b = s1x[sc]
            a.wait_recv()
            b.wait_recv()
            c2_my[sc] = (part_ref[0, rows_v(sc), :].astype(F32)
                         + r1x[0, rows_h(sc), :].astype(F32))
            c2buf[rows_h(sc), :] = (part_ref[1, rows_v(sc), :].astype(F32)
                                    + r1x[1, rows_h(sc), :].astype(F32)).astype(BF16)
            s2y[sc] = xchg(c2buf.at[rows_h(sc), :], r2y.at[rows_h(sc), :], py)

        s3x, s3y = [None] * SC, [None] * SC
        for sc in range(SC):
            s2x[sc].wait_recv()
            fu = silu(c1_my[sc] + r2x[rows_h(sc), :].astype(F32)).astype(BF16)
            finu[rows_h(sc), :] = fu
            out_ref[rows_u(sc), pl.ds(my * nb, nb)] = fu
            s3x[sc] = xchg(finu.at[rows_h(sc), :], r3x.at[rows_h(sc), :], px)

            s2y[sc].wait_recv()
            fv = silu(c2_my[sc] + r2y[rows_h(sc), :].astype(F32)).astype(BF16)
            finv[rows_h(sc), :] = fv
            out_ref[rows_v(sc), pl.ds(my * nb, nb)] = fv
            s3y[sc] = xchg(finv.at[rows_h(sc), :], r3y.at[rows_h(sc), :], py)

        s4y, s4x = [None] * SC, [None] * SC
        for sc in range(SC):
            s3x[sc].wait_recv()
            out_ref[rows_u(sc), pl.ds(px * nb, nb)] = r3x[rows_h(sc), :]
            s4y[sc] = (
                xchg(finu.at[rows_h(sc), :], r4y.at[0, rows_h(sc), :], py),
                xchg(r3x.at[rows_h(sc), :], r4y.at[1, rows_h(sc), :], py),
            )

            s3y[sc].wait_recv()
            out_ref[rows_v(sc), pl.ds(py * nb, nb)] = r3y[rows_h(sc), :]
            s4x[sc] = (
                xchg(finv.at[rows_h(sc), :], r4x.at[0, rows_h(sc), :], px),
                xchg(r3y.at[rows_h(sc), :], r4x.at[1, rows_h(sc), :], px),
            )

        for sc in range(SC):
            a, b = s4y[sc]
            a.wait_recv()
            out_ref[rows_u(sc), pl.ds(py * nb, nb)] = r4y[0, rows_h(sc), :]
            b.wait_recv()
            out_ref[rows_u(sc), pl.ds(dg * nb, nb)] = r4y[1, rows_h(sc), :]
            a, b = s4x[sc]
            a.wait_recv()
            out_ref[rows_v(sc), pl.ds(px * nb, nb)] = r4x[0, rows_h(sc), :]
            b.wait_recv()
            out_ref[rows_v(sc), pl.ds(dg * nb, nb)] = r4x[1, rows_h(sc), :]

        for r in rdmas:
            r.wait_send()

    return pl.pallas_call(
        body,
        out_shape=jax.ShapeDtypeStruct((m, n), BF16),
        in_specs=[
            pl.BlockSpec(memory_space=pltpu.VMEM),
            pl.BlockSpec(memory_space=pltpu.VMEM),
        ],
        out_specs=pl.BlockSpec(memory_space=pltpu.VMEM),
        scratch_shapes=[
            pltpu.VMEM((N_DEV, m, nb), BF16),
            pltpu.VMEM((2, mh, nb), BF16),
            pltpu.VMEM((2, mh, nb), BF16),
            pltpu.VMEM((mh, nb), BF16),
            pltpu.VMEM((mh, nb), BF16),
            pltpu.VMEM((mh, nb), BF16),
            pltpu.VMEM((mh, nb), BF16),
            pltpu.VMEM((mh, nb), BF16),
            pltpu.VMEM((mh, nb), BF16),
            pltpu.VMEM((mh, nb), BF16),
            pltpu.VMEM((mh, nb), BF16),
            pltpu.VMEM((2, mh, nb), BF16),
            pltpu.VMEM((2, mh, nb), BF16),
            pltpu.SemaphoreType.DMA((12 * SC,)),
            pltpu.SemaphoreType.DMA((12 * SC,)),
        ],
        compiler_params=pltpu.CompilerParams(collective_id=0),
    )(A.astype(BF16), B.astype(BF16))


# device time: 27028 ns/iter; 1.6710x vs baseline; 1.0002x over previous
import jax
import jax.numpy as jnp
from jax import lax
from jax.experimental import pallas as pl
from jax.experimental.pallas import tpu as pltpu

N_DEV = 4
SC = 4
F32 = jnp.float32
BF16 = jnp.bfloat16


def kernel(A, B):
    m, _ = A.shape
    _, n = B.shape
    nb = n // N_DEV
    mh = m // 2
    qr = mh // SC

    def rows_u(sc):
        return pl.ds(sc * qr, qr)

    def rows_v(sc):
        return pl.ds(mh + sc * qr, qr)

    def rows_h(sc):
        return pl.ds(sc * qr, qr)

    def body(a_ref, b_ref, out_ref, part_ref,
             r1y, r1x, c1buf, c2buf, r2x, r2y,
             finu, finv, r3x, r3y, r4y, r4x,
             send_sems, recv_sems):
        my = lax.axis_index("i")
        py = my ^ 1
        px = my ^ 3
        dg = my ^ 2

        sem_i = iter(range(12 * SC))
        rdmas = []

        def xchg(src, dst, peer):
            i = next(sem_i)
            rdma = pltpu.make_async_remote_copy(
                src_ref=src, dst_ref=dst,
                send_sem=send_sems.at[i], recv_sem=recv_sems.at[i],
                device_id=(peer,), device_id_type=pl.DeviceIdType.MESH,
            )
            rdma.start()
            rdmas.append(rdma)
            return rdma

        barrier_sem = pltpu.get_barrier_semaphore()
        for nbr in (py, px):
            pl.semaphore_signal(
                barrier_sem, inc=1,
                device_id=(nbr,), device_id_type=pl.DeviceIdType.MESH,
            )

        UH, VH = pl.ds(0, mh), pl.ds(mh, mh)

        def pblock_h(g, slot, half):
            b_blk = b_ref[:, pl.ds(g * nb, nb)]
            part_ref[slot, half, :] = jnp.dot(
                a_ref[half, :], b_blk, preferred_element_type=F32
            ).astype(BF16)

        pblock_h(py, 1, UH)
        pl.semaphore_wait(barrier_sem, 2)
        s1y = [[None, None] for _ in range(SC)]
        s1x = [[None, None] for _ in range(SC)]
        s1y[0][0] = xchg(part_ref.at[1, rows_u(0), :], r1y.at[0, rows_h(0), :], py)
        pblock_h(px, 3, VH)
        s1x[0][0] = xchg(part_ref.at[3, rows_v(0), :], r1x.at[0, rows_h(0), :], px)
        pblock_h(dg, 2, UH)
        s1y[0][1] = xchg(part_ref.at[2, rows_u(0), :], r1y.at[1, rows_h(0), :], py)
        for sc in range(1, SC):
            s1y[sc][0] = xchg(part_ref.at[1, rows_u(sc), :], r1y.at[0, rows_h(sc), :], py)
            s1y[sc][1] = xchg(part_ref.at[2, rows_u(sc), :], r1y.at[1, rows_h(sc), :], py)
        pblock_h(dg, 2, VH)
        s1x[0][1] = xchg(part_ref.at[2, rows_v(0), :], r1x.at[1, rows_h(0), :], px)
        for sc in range(1, SC):
            s1x[sc][0] = xchg(part_ref.at[3, rows_v(sc), :], r1x.at[0, rows_h(sc), :], px)
            s1x[sc][1] = xchg(part_ref.at[2, rows_v(sc), :], r1x.at[1, rows_h(sc), :], px)
        pblock_h(px, 3, UH)
        pblock_h(py, 1, VH)
        pblock_h(my, 0, UH)
        pblock_h(my, 0, VH)

        def silu(z):
            return z * (1.0 / (1.0 + jnp.exp(-z)))

        c1_my, c2_my, s2x, s2y = [None] * SC, [None] * SC, [None] * SC, [None] * SC
        for sc in range(SC):
            a, b = s1y[sc]
            a.wait_recv()
            b.wait_recv()
            c1_my[sc] = (part_ref[0, rows_u(sc), :].astype(F32)
                         + r1y[0, rows_h(sc), :].astype(F32))
            c1buf[rows_h(sc), :] = (part_ref[3, rows_u(sc), :].astype(F32)
                                    + r1y[1, rows_h(sc), :].astype(F32)).astype(BF16)
            s2x[sc] = xchg(c1buf.at[rows_h(sc), :], r2x.at[rows_h(sc), :], px)

            a, b = s1x[sc]
            a.wait_recv()
            b.wait_recv()
            c2_my[sc] = (part_ref[0, rows_v(sc), :].astype(F32)
                         + r1x[0, rows_h(sc), :].astype(F32))
            c2buf[rows_h(sc), :] = (part_ref[1, rows_v(sc), :].astype(F32)
                                    + r1x[1, rows_h(sc), :].astype(F32)).astype(BF16)
            s2y[sc] = xchg(c2buf.at[rows_h(sc), :], r2y.at[rows_h(sc), :], py)

        s3x, s3y = [None] * SC, [None] * SC
        s4ya, s4yb = [None] * SC, [None] * SC
        s4xa, s4xb = [None] * SC, [None] * SC
        for sc in range(SC):
            s2x[sc].wait_recv()
            fu = silu(c1_my[sc] + r2x[rows_h(sc), :].astype(F32)).astype(BF16)
            finu[rows_h(sc), :] = fu
            out_ref[rows_u(sc), pl.ds(my * nb, nb)] = fu
            s3x[sc] = xchg(finu.at[rows_h(sc), :], r3x.at[rows_h(sc), :], px)
            s4ya[sc] = xchg(finu.at[rows_h(sc), :], r4y.at[0, rows_h(sc), :], py)

            s2y[sc].wait_recv()
            fv = silu(c2_my[sc] + r2y[rows_h(sc), :].astype(F32)).astype(BF16)
            finv[rows_h(sc), :] = fv
            out_ref[rows_v(sc), pl.ds(my * nb, nb)] = fv
            s3y[sc] = xchg(finv.at[rows_h(sc), :], r3y.at[rows_h(sc), :], py)
            s4xa[sc] = xchg(finv.at[rows_h(sc), :], r4x.at[0, rows_h(sc), :], px)

        for sc in range(SC):
            s3x[sc].wait_recv()
            out_ref[rows_u(sc), pl.ds(px * nb, nb)] = r3x[rows_h(sc), :]
            s4yb[sc] = xchg(r3x.at[rows_h(sc), :], r4y.at[1, rows_h(sc), :], py)

            s3y[sc].wait_recv()
            out_ref[rows_v(sc), pl.ds(py * nb, nb)] = r3y[rows_h(sc), :]
            s4xb[sc] = xchg(r3y.at[rows_h(sc), :], r4x.at[1, rows_h(sc), :], px)

        for sc in range(SC):
            s4ya[sc].wait_recv()
            out_ref[rows_u(sc), pl.ds(py * nb, nb)] = r4y[0, rows_h(sc), :]
            s4xa[sc].wait_recv()
            out_ref[rows_v(sc), pl.ds(px * nb, nb)] = r4x[0, rows_h(sc), :]
            s4yb[sc].wait_recv()
            out_ref[rows_u(sc), pl.ds(dg * nb, nb)] = r4y[1, rows_h(sc), :]
            s4xb[sc].wait_recv()
            out_ref[rows_v(sc), pl.ds(dg * nb, nb)] = r4x[1, rows_h(sc), :]

        for r in rdmas:
            r.wait_send()

    return pl.pallas_call(
        body,
        out_shape=jax.ShapeDtypeStruct((m, n), BF16),
        in_specs=[
            pl.BlockSpec(memory_space=pltpu.VMEM),
            pl.BlockSpec(memory_space=pltpu.VMEM),
        ],
        out_specs=pl.BlockSpec(memory_space=pltpu.VMEM),
        scratch_shapes=[
            pltpu.VMEM((N_DEV, m, nb), BF16),
            pltpu.VMEM((2, mh, nb), BF16),
            pltpu.VMEM((2, mh, nb), BF16),
            pltpu.VMEM((mh, nb), BF16),
            pltpu.VMEM((mh, nb), BF16),
            pltpu.VMEM((mh, nb), BF16),
            pltpu.VMEM((mh, nb), BF16),
            pltpu.VMEM((mh, nb), BF16),
            pltpu.VMEM((mh, nb), BF16),
            pltpu.VMEM((mh, nb), BF16),
            pltpu.VMEM((mh, nb), BF16),
            pltpu.VMEM((2, mh, nb), BF16),
            pltpu.VMEM((2, mh, nb), BF16),
            pltpu.SemaphoreType.DMA((12 * SC,)),
            pltpu.SemaphoreType.DMA((12 * SC,)),
        ],
        compiler_params=pltpu.CompilerParams(collective_id=0),
    )(A.astype(BF16), B.astype(BF16))
